# baseline (speedup 1.0000x reference)
import functools

import jax
import jax.numpy as jnp
from jax import lax
from jax.experimental import pallas as pl
from jax.experimental.pallas import tpu as pltpu
from jax.experimental.pallas import tpu_sc as plsc

_SPACING = 0.001
_SHIFT = _SPACING / 2.0

_N_IN = 4096
_C = 128
_N_NEW = 4096

_Q_TILE = 128
_CHUNK = 128


def _argmin_body(q_ref, c_ref, idx_ref):
    shape = (_Q_TILE, _CHUNK)
    qx = jnp.broadcast_to(q_ref[:, 0:1], shape)
    qy = jnp.broadcast_to(q_ref[:, 1:2], shape)
    nstream = 4
    nch = _N_IN // _CHUNK
    best = [jnp.full(shape, jnp.inf, jnp.float32) for _ in range(nstream)]
    bchunk = [jnp.zeros(shape, jnp.int32) for _ in range(nstream)]
    for ch in range(nch):
        s = ch % nstream
        cx = c_ref[0:1, pl.ds(ch * _CHUNK, _CHUNK)]
        cy = c_ref[1:2, pl.ds(ch * _CHUNK, _CHUNK)]
        dx = qx - cx
        dy = qy - cy
        d2 = dx * dx + dy * dy
        lt = d2 < best[s]
        best[s] = jnp.where(lt, d2, best[s])
        bchunk[s] = jnp.where(lt, ch, bchunk[s])
    lane = lax.broadcasted_iota(jnp.int32, shape, 1)
    bval, bflat = best[0], bchunk[0] * _CHUNK + lane
    for s in range(1, nstream):
        v, f = best[s], bchunk[s] * _CHUNK + lane
        take = (v < bval) | ((v == bval) & (f < bflat))
        bval = jnp.where(take, v, bval)
        bflat = jnp.where(take, f, bflat)
    m = jnp.min(bval, axis=1, keepdims=True)
    res = jnp.min(jnp.where(bval == m, bflat, _N_IN), axis=1)
    idx_ref[...] = res.reshape(1, 1, _Q_TILE)


def _nn_argmin(q, coords_t, interpret=False):
    grid = _N_NEW // _Q_TILE
    return pl.pallas_call(
        _argmin_body,
        grid=(grid,),
        in_specs=[
            pl.BlockSpec((_Q_TILE, 2), lambda i: (i, 0)),
            pl.BlockSpec((2, _N_IN), lambda i: (0, 0)),
        ],
        out_specs=pl.BlockSpec((1, 1, _Q_TILE), lambda i: (i, 0, 0)),
        out_shape=jax.ShapeDtypeStruct((grid, 1, _Q_TILE), jnp.int32),
        interpret=interpret,
    )(q, coords_t).reshape(_N_NEW)


_NC, _NS = 2, 16
_NW = _NC * _NS
_R_PER_W = _C // _NW
_L = 16


def _gather_body(values_hbm, idx_hbm, out_hbm, idx_v, rows_v, new_v, sem, lsem):
    wid = lax.axis_index("s") * _NC + lax.axis_index("c")
    row0 = wid * _R_PER_W
    left = [
        pltpu.make_async_copy(
            values_hbm.at[row0 + r], out_hbm.at[row0 + r, pl.ds(0, _N_IN)], lsem
        )
        for r in range(_R_PER_W)
    ]
    for cp in left:
        cp.start()
    copies = [pltpu.make_async_copy(idx_hbm, idx_v, sem)]
    copies += [
        pltpu.make_async_copy(
            values_hbm.at[row0 + r], rows_v.at[pl.ds(r * _N_IN, _N_IN)], sem
        )
        for r in range(_R_PER_W)
    ]
    for cp in copies:
        cp.start()
    for cp in copies:
        cp.wait()

    outs = []
    for r in range(_R_PER_W):

        def body(k, carry, r=r):
            ich = idx_v[pl.ds(k * _L, _L)]
            g = plsc.load_gather(rows_v, [ich + (r * _N_IN)])
            new_v[pl.ds(r * _N_IN + k * _L, _L)] = g
            return carry

        lax.fori_loop(0, _N_IN // _L, body, 0)
        cp = pltpu.make_async_copy(
            new_v.at[pl.ds(r * _N_IN, _N_IN)],
            out_hbm.at[row0 + r, pl.ds(_N_IN, _N_IN)],
            sem,
        )
        cp.start()
        outs.append(cp)
    for cp in outs:
        cp.wait()
    for cp in left:
        cp.wait()


@functools.cache
def _make_gather():
    return pl.kernel(
        _gather_body,
        out_type=jax.ShapeDtypeStruct((_C, 2 * _N_IN), jnp.float32),
        mesh=plsc.VectorSubcoreMesh(
            core_axis_name="c", subcore_axis_name="s", num_cores=_NC
        ),
        scratch_types=[
            pltpu.VMEM((_N_NEW,), jnp.int32),
            pltpu.VMEM((_R_PER_W * _N_IN,), jnp.float32),
            pltpu.VMEM((_R_PER_W * _N_IN,), jnp.float32),
            pltpu.SemaphoreType.DMA,
            pltpu.SemaphoreType.DMA,
        ],
        compiler_params=pltpu.CompilerParams(needs_layout_passes=False),
    )


@jax.jit
def kernel(coords, values, dropped_coords):
    q = dropped_coords - _SHIFT
    nn_idx = _nn_argmin(q, coords.T)
    return _make_gather()(values, nn_idx)


# speedup vs baseline: 1.9416x; 1.9416x over previous
import functools

import jax
import jax.numpy as jnp
from jax import lax
from jax.experimental import pallas as pl
from jax.experimental.pallas import tpu as pltpu
from jax.experimental.pallas import tpu_sc as plsc

_SPACING = 0.001
_SHIFT = _SPACING / 2.0

_N_IN = 4096
_C = 128
_N_NEW = 4096

_Q_TILE = 128
_CHUNK = 128


def _argmin_body(q_ref, c_ref, idx_ref):
    shape = (_Q_TILE, _CHUNK)
    qx = jnp.broadcast_to(q_ref[:, 0:1], shape)
    qy = jnp.broadcast_to(q_ref[:, 1:2], shape)
    nstream = 4
    nch = _N_IN // _CHUNK
    best = [jnp.full(shape, jnp.inf, jnp.float32) for _ in range(nstream)]
    bchunk = [jnp.zeros(shape, jnp.int32) for _ in range(nstream)]
    for ch in range(nch):
        s = ch % nstream
        cx = c_ref[0:1, pl.ds(ch * _CHUNK, _CHUNK)]
        cy = c_ref[1:2, pl.ds(ch * _CHUNK, _CHUNK)]
        dx = qx - cx
        dy = qy - cy
        d2 = dx * dx + dy * dy
        lt = d2 < best[s]
        best[s] = jnp.where(lt, d2, best[s])
        bchunk[s] = jnp.where(lt, ch, bchunk[s])
    lane = lax.broadcasted_iota(jnp.int32, shape, 1)
    bval, bflat = best[0], bchunk[0] * _CHUNK + lane
    for s in range(1, nstream):
        v, f = best[s], bchunk[s] * _CHUNK + lane
        take = (v < bval) | ((v == bval) & (f < bflat))
        bval = jnp.where(take, v, bval)
        bflat = jnp.where(take, f, bflat)
    m = jnp.min(bval, axis=1, keepdims=True)
    res = jnp.min(jnp.where(bval == m, bflat, _N_IN), axis=1)
    idx_ref[...] = res.reshape(1, 1, _Q_TILE)


def _nn_argmin(q, coords_t, interpret=False):
    grid = _N_NEW // _Q_TILE
    return pl.pallas_call(
        _argmin_body,
        grid=(grid,),
        in_specs=[
            pl.BlockSpec((_Q_TILE, 2), lambda i: (i, 0)),
            pl.BlockSpec((2, _N_IN), lambda i: (0, 0)),
        ],
        out_specs=pl.BlockSpec((1, 1, _Q_TILE), lambda i: (i, 0, 0)),
        out_shape=jax.ShapeDtypeStruct((grid, 1, _Q_TILE), jnp.int32),
        interpret=interpret,
    )(q, coords_t).reshape(_N_NEW)


_NC, _NS = 2, 16
_NW = _NC * _NS
_R_PER_W = _C // _NW
_L = 16


def _gather_body(values_hbm, idx_hbm, out_hbm, idx_v, rows_v, new_v, sem, lsem):
    wid = lax.axis_index("s") * _NC + lax.axis_index("c")
    row0 = wid * _R_PER_W
    copies = [pltpu.make_async_copy(idx_hbm, idx_v, sem)]
    copies += [
        pltpu.make_async_copy(
            values_hbm.at[row0 + r], rows_v.at[pl.ds(r * _N_IN, _N_IN)], sem
        )
        for r in range(_R_PER_W)
    ]
    for cp in copies:
        cp.start()
    for cp in copies:
        cp.wait()

    left = [
        pltpu.make_async_copy(
            rows_v.at[pl.ds(r * _N_IN, _N_IN)],
            out_hbm.at[row0 + r, pl.ds(0, _N_IN)],
            lsem,
        )
        for r in range(_R_PER_W)
    ]
    for cp in left:
        cp.start()

    def body(k, carry):
        ich = idx_v[pl.ds(k * _L, _L)]
        for r in range(_R_PER_W):
            g = plsc.load_gather(rows_v, [ich + (r * _N_IN)])
            new_v[pl.ds(r * _N_IN + k * _L, _L)] = g
        return carry

    lax.fori_loop(0, _N_IN // _L, body, 0)

    outs = [
        pltpu.make_async_copy(
            new_v.at[pl.ds(r * _N_IN, _N_IN)],
            out_hbm.at[row0 + r, pl.ds(_N_IN, _N_IN)],
            sem,
        )
        for r in range(_R_PER_W)
    ]
    for cp in outs:
        cp.start()
    for cp in outs:
        cp.wait()
    for cp in left:
        cp.wait()


@functools.cache
def _make_gather():
    return pl.kernel(
        _gather_body,
        out_type=jax.ShapeDtypeStruct((_C, 2 * _N_IN), jnp.float32),
        mesh=plsc.VectorSubcoreMesh(
            core_axis_name="c", subcore_axis_name="s", num_cores=_NC
        ),
        scratch_types=[
            pltpu.VMEM((_N_NEW,), jnp.int32),
            pltpu.VMEM((_R_PER_W * _N_IN,), jnp.float32),
            pltpu.VMEM((_R_PER_W * _N_IN,), jnp.float32),
            pltpu.SemaphoreType.DMA,
            pltpu.SemaphoreType.DMA,
        ],
        compiler_params=pltpu.CompilerParams(needs_layout_passes=False),
    )


@jax.jit
def kernel(coords, values, dropped_coords):
    q = dropped_coords - _SHIFT
    nn_idx = _nn_argmin(q, coords.T)
    return _make_gather()(values, nn_idx)


# speedup vs baseline: 2.1479x; 1.1062x over previous
import functools

import jax
import jax.numpy as jnp
from jax import lax
from jax.experimental import pallas as pl
from jax.experimental.pallas import tpu as pltpu
from jax.experimental.pallas import tpu_sc as plsc

_SPACING = 0.001
_SHIFT = _SPACING / 2.0

_N_IN = 4096
_C = 128
_N_NEW = 4096

_NG = 4
_UNROLL = 8


def _argmin_body(qx_ref, qy_ref, c_ref, idx_ref):
    shape = (8, 128)
    qx = [qx_ref[g] for g in range(_NG)]
    qy = [qy_ref[g] for g in range(_NG)]
    inf = jnp.full(shape, jnp.inf, jnp.float32)
    zero = jnp.zeros(shape, jnp.int32)
    state = ([inf] * _NG, [zero] * _NG, [inf] * _NG, [zero] * _NG)

    def body(k, st):
        best_a, bidx_a, best_b, bidx_b = st
        best = [list(best_a), list(best_b)]
        bidx = [list(bidx_a), list(bidx_b)]
        j0 = k * _UNROLL
        for u in range(_UNROLL):
            s = u % 2
            j = j0 + u
            cx = jnp.full(shape, c_ref[0, j])
            cy = jnp.full(shape, c_ref[1, j])
            for g in range(_NG):
                dx = qx[g] - cx
                dy = qy[g] - cy
                d2 = dx * dx + dy * dy
                lt = d2 < best[s][g]
                best[s][g] = jnp.where(lt, d2, best[s][g])
                bidx[s][g] = jnp.where(lt, j, bidx[s][g])
        return (best[0], bidx[0], best[1], bidx[1])

    best_a, bidx_a, best_b, bidx_b = lax.fori_loop(
        0, _N_IN // _UNROLL, body, state
    )
    for g in range(_NG):
        take = (best_b[g] < best_a[g]) | (
            (best_b[g] == best_a[g]) & (bidx_b[g] < bidx_a[g])
        )
        idx_ref[g] = jnp.where(take, bidx_b[g], bidx_a[g])


def _nn_argmin(qx, qy, coords_t, interpret=False):
    return pl.pallas_call(
        _argmin_body,
        in_specs=[
            pl.BlockSpec(memory_space=pltpu.VMEM),
            pl.BlockSpec(memory_space=pltpu.VMEM),
            pl.BlockSpec(memory_space=pltpu.SMEM),
        ],
        out_specs=pl.BlockSpec(memory_space=pltpu.VMEM),
        out_shape=jax.ShapeDtypeStruct((_NG, 8, 128), jnp.int32),
        interpret=interpret,
    )(qx, qy, coords_t).reshape(_N_NEW)


_NC, _NS = 2, 16
_NW = _NC * _NS
_R_PER_W = _C // _NW
_L = 16


def _gather_body(values_hbm, idx_hbm, out_hbm, idx_v, rows_v, new_v, sem, lsem):
    wid = lax.axis_index("s") * _NC + lax.axis_index("c")
    row0 = wid * _R_PER_W
    copies = [pltpu.make_async_copy(idx_hbm, idx_v, sem)]
    copies += [
        pltpu.make_async_copy(
            values_hbm.at[row0 + r], rows_v.at[pl.ds(r * _N_IN, _N_IN)], sem
        )
        for r in range(_R_PER_W)
    ]
    for cp in copies:
        cp.start()
    for cp in copies:
        cp.wait()

    left = [
        pltpu.make_async_copy(
            rows_v.at[pl.ds(r * _N_IN, _N_IN)],
            out_hbm.at[row0 + r, pl.ds(0, _N_IN)],
            lsem,
        )
        for r in range(_R_PER_W)
    ]
    for cp in left:
        cp.start()

    def body(k, carry):
        ich = idx_v[pl.ds(k * _L, _L)]
        for r in range(_R_PER_W):
            g = plsc.load_gather(rows_v, [ich + (r * _N_IN)])
            new_v[pl.ds(r * _N_IN + k * _L, _L)] = g
        return carry

    lax.fori_loop(0, _N_IN // _L, body, 0)

    outs = [
        pltpu.make_async_copy(
            new_v.at[pl.ds(r * _N_IN, _N_IN)],
            out_hbm.at[row0 + r, pl.ds(_N_IN, _N_IN)],
            sem,
        )
        for r in range(_R_PER_W)
    ]
    for cp in outs:
        cp.start()
    for cp in outs:
        cp.wait()
    for cp in left:
        cp.wait()


@functools.cache
def _make_gather():
    return pl.kernel(
        _gather_body,
        out_type=jax.ShapeDtypeStruct((_C, 2 * _N_IN), jnp.float32),
        mesh=plsc.VectorSubcoreMesh(
            core_axis_name="c", subcore_axis_name="s", num_cores=_NC
        ),
        scratch_types=[
            pltpu.VMEM((_N_NEW,), jnp.int32),
            pltpu.VMEM((_R_PER_W * _N_IN,), jnp.float32),
            pltpu.VMEM((_R_PER_W * _N_IN,), jnp.float32),
            pltpu.SemaphoreType.DMA,
            pltpu.SemaphoreType.DMA,
        ],
        compiler_params=pltpu.CompilerParams(needs_layout_passes=False),
    )


@jax.jit
def kernel(coords, values, dropped_coords):
    q = dropped_coords - _SHIFT
    qx = q[:, 0].reshape(_NG, 8, 128)
    qy = q[:, 1].reshape(_NG, 8, 128)
    nn_idx = _nn_argmin(qx, qy, coords.T)
    return _make_gather()(values, nn_idx)


# speedup vs baseline: 2.2453x; 1.0453x over previous
import functools

import jax
import jax.numpy as jnp
from jax import lax
from jax.experimental import pallas as pl
from jax.experimental.pallas import tpu as pltpu
from jax.experimental.pallas import tpu_sc as plsc

_SPACING = 0.001
_SHIFT = _SPACING / 2.0

_N_IN = 4096
_C = 128
_N_NEW = 4096

_NG = 4
_UNROLL = 16


def _argmin_body(qx_ref, qy_ref, c_ref, idx_ref):
    shape = (8, 128)
    qx = [qx_ref[g] for g in range(_NG)]
    qy = [qy_ref[g] for g in range(_NG)]
    inf = jnp.full(shape, jnp.inf, jnp.float32)
    zero = jnp.zeros(shape, jnp.int32)
    state = ([inf] * _NG, [zero] * _NG, [inf] * _NG, [zero] * _NG)

    def body(k, st):
        best_a, bidx_a, best_b, bidx_b = st
        best = [list(best_a), list(best_b)]
        bidx = [list(bidx_a), list(bidx_b)]
        j0 = k * _UNROLL
        for u in range(_UNROLL):
            s = u % 2
            j = j0 + u
            cx = jnp.full(shape, c_ref[0, j])
            cy = jnp.full(shape, c_ref[1, j])
            for g in range(_NG):
                dx = qx[g] - cx
                dy = qy[g] - cy
                d2 = dx * dx + dy * dy
                lt = d2 < best[s][g]
                best[s][g] = jnp.where(lt, d2, best[s][g])
                bidx[s][g] = jnp.where(lt, j, bidx[s][g])
        return (best[0], bidx[0], best[1], bidx[1])

    best_a, bidx_a, best_b, bidx_b = lax.fori_loop(
        0, _N_IN // _UNROLL, body, state
    )
    for g in range(_NG):
        take = (best_b[g] < best_a[g]) | (
            (best_b[g] == best_a[g]) & (bidx_b[g] < bidx_a[g])
        )
        idx_ref[g] = jnp.where(take, bidx_b[g], bidx_a[g])


def _nn_argmin(qx, qy, coords_t, interpret=False):
    return pl.pallas_call(
        _argmin_body,
        in_specs=[
            pl.BlockSpec(memory_space=pltpu.VMEM),
            pl.BlockSpec(memory_space=pltpu.VMEM),
            pl.BlockSpec(memory_space=pltpu.SMEM),
        ],
        out_specs=pl.BlockSpec(memory_space=pltpu.VMEM),
        out_shape=jax.ShapeDtypeStruct((_NG, 8, 128), jnp.int32),
        interpret=interpret,
    )(qx, qy, coords_t).reshape(_N_NEW)


_NC, _NS = 2, 16
_NW = _NC * _NS
_R_PER_W = _C // _NW
_L = 16


def _gather_body(values_hbm, idx_hbm, out_hbm, idx_v, rows_v, new_v, sem, lsem):
    wid = lax.axis_index("s") * _NC + lax.axis_index("c")
    row0 = wid * _R_PER_W
    copies = [pltpu.make_async_copy(idx_hbm, idx_v, sem)]
    copies += [
        pltpu.make_async_copy(
            values_hbm.at[row0 + r], rows_v.at[pl.ds(r * _N_IN, _N_IN)], sem
        )
        for r in range(_R_PER_W)
    ]
    for cp in copies:
        cp.start()
    for cp in copies:
        cp.wait()

    left = [
        pltpu.make_async_copy(
            rows_v.at[pl.ds(r * _N_IN, _N_IN)],
            out_hbm.at[row0 + r, pl.ds(0, _N_IN)],
            lsem,
        )
        for r in range(_R_PER_W)
    ]
    for cp in left:
        cp.start()

    def body(k, carry):
        ich = idx_v[pl.ds(k * _L, _L)]
        for r in range(_R_PER_W):
            g = plsc.load_gather(rows_v, [ich + (r * _N_IN)])
            new_v[pl.ds(r * _N_IN + k * _L, _L)] = g
        return carry

    lax.fori_loop(0, _N_IN // _L, body, 0)

    outs = [
        pltpu.make_async_copy(
            new_v.at[pl.ds(r * _N_IN, _N_IN)],
            out_hbm.at[row0 + r, pl.ds(_N_IN, _N_IN)],
            sem,
        )
        for r in range(_R_PER_W)
    ]
    for cp in outs:
        cp.start()
    for cp in outs:
        cp.wait()
    for cp in left:
        cp.wait()


@functools.cache
def _make_gather():
    return pl.kernel(
        _gather_body,
        out_type=jax.ShapeDtypeStruct((_C, 2 * _N_IN), jnp.float32),
        mesh=plsc.VectorSubcoreMesh(
            core_axis_name="c", subcore_axis_name="s", num_cores=_NC
        ),
        scratch_types=[
            pltpu.VMEM((_N_NEW,), jnp.int32),
            pltpu.VMEM((_R_PER_W * _N_IN,), jnp.float32),
            pltpu.VMEM((_R_PER_W * _N_IN,), jnp.float32),
            pltpu.SemaphoreType.DMA,
            pltpu.SemaphoreType.DMA,
        ],
        compiler_params=pltpu.CompilerParams(needs_layout_passes=False),
    )


@jax.jit
def kernel(coords, values, dropped_coords):
    q = dropped_coords - _SHIFT
    qx = q[:, 0].reshape(_NG, 8, 128)
    qy = q[:, 1].reshape(_NG, 8, 128)
    nn_idx = _nn_argmin(qx, qy, coords.T)
    return _make_gather()(values, nn_idx)


# speedup vs baseline: 2.2856x; 1.0180x over previous
import functools

import jax
import jax.numpy as jnp
from jax import lax
from jax.experimental import pallas as pl
from jax.experimental.pallas import tpu as pltpu
from jax.experimental.pallas import tpu_sc as plsc

_SPACING = 0.001
_SHIFT = _SPACING / 2.0

_N_IN = 4096
_C = 128
_N_NEW = 4096

_NG = 4
_UNROLL = 32


def _argmin_body(qx_ref, qy_ref, c_ref, idx_ref):
    shape = (8, 128)
    qx = [qx_ref[g] for g in range(_NG)]
    qy = [qy_ref[g] for g in range(_NG)]
    inf = jnp.full(shape, jnp.inf, jnp.float32)
    zero = jnp.zeros(shape, jnp.int32)
    state = ([inf] * _NG, [zero] * _NG, [inf] * _NG, [zero] * _NG)

    def body(k, st):
        best_a, bidx_a, best_b, bidx_b = st
        best = [list(best_a), list(best_b)]
        bidx = [list(bidx_a), list(bidx_b)]
        j0 = k * _UNROLL
        for u in range(_UNROLL):
            s = u % 2
            j = j0 + u
            cx = jnp.full(shape, c_ref[0, j])
            cy = jnp.full(shape, c_ref[1, j])
            for g in range(_NG):
                dx = qx[g] - cx
                dy = qy[g] - cy
                d2 = dx * dx + dy * dy
                lt = d2 < best[s][g]
                best[s][g] = jnp.where(lt, d2, best[s][g])
                bidx[s][g] = jnp.where(lt, j, bidx[s][g])
        return (best[0], bidx[0], best[1], bidx[1])

    best_a, bidx_a, best_b, bidx_b = lax.fori_loop(
        0, _N_IN // _UNROLL, body, state
    )
    for g in range(_NG):
        take = (best_b[g] < best_a[g]) | (
            (best_b[g] == best_a[g]) & (bidx_b[g] < bidx_a[g])
        )
        idx_ref[g] = jnp.where(take, bidx_b[g], bidx_a[g])


def _nn_argmin(qx, qy, coords_t, interpret=False):
    return pl.pallas_call(
        _argmin_body,
        in_specs=[
            pl.BlockSpec(memory_space=pltpu.VMEM),
            pl.BlockSpec(memory_space=pltpu.VMEM),
            pl.BlockSpec(memory_space=pltpu.SMEM),
        ],
        out_specs=pl.BlockSpec(memory_space=pltpu.VMEM),
        out_shape=jax.ShapeDtypeStruct((_NG, 8, 128), jnp.int32),
        interpret=interpret,
    )(qx, qy, coords_t).reshape(_N_NEW)


_NC, _NS = 2, 16
_NW = _NC * _NS
_R_PER_W = _C // _NW
_L = 16


def _gather_body(values_hbm, idx_hbm, out_hbm, idx_v, rows_v, new_v, sem, lsem):
    wid = lax.axis_index("s") * _NC + lax.axis_index("c")
    row0 = wid * _R_PER_W
    copies = [pltpu.make_async_copy(idx_hbm, idx_v, sem)]
    copies += [
        pltpu.make_async_copy(
            values_hbm.at[row0 + r], rows_v.at[pl.ds(r * _N_IN, _N_IN)], sem
        )
        for r in range(_R_PER_W)
    ]
    for cp in copies:
        cp.start()
    for cp in copies:
        cp.wait()

    left = [
        pltpu.make_async_copy(
            rows_v.at[pl.ds(r * _N_IN, _N_IN)],
            out_hbm.at[row0 + r, pl.ds(0, _N_IN)],
            lsem,
        )
        for r in range(_R_PER_W)
    ]
    for cp in left:
        cp.start()

    def body(k, carry):
        ich = idx_v[pl.ds(k * _L, _L)]
        for r in range(_R_PER_W):
            g = plsc.load_gather(rows_v, [ich + (r * _N_IN)])
            new_v[pl.ds(r * _N_IN + k * _L, _L)] = g
        return carry

    lax.fori_loop(0, _N_IN // _L, body, 0)

    outs = [
        pltpu.make_async_copy(
            new_v.at[pl.ds(r * _N_IN, _N_IN)],
            out_hbm.at[row0 + r, pl.ds(_N_IN, _N_IN)],
            sem,
        )
        for r in range(_R_PER_W)
    ]
    for cp in outs:
        cp.start()
    for cp in outs:
        cp.wait()
    for cp in left:
        cp.wait()


@functools.cache
def _make_gather():
    return pl.kernel(
        _gather_body,
        out_type=jax.ShapeDtypeStruct((_C, 2 * _N_IN), jnp.float32),
        mesh=plsc.VectorSubcoreMesh(
            core_axis_name="c", subcore_axis_name="s", num_cores=_NC
        ),
        scratch_types=[
            pltpu.VMEM((_N_NEW,), jnp.int32),
            pltpu.VMEM((_R_PER_W * _N_IN,), jnp.float32),
            pltpu.VMEM((_R_PER_W * _N_IN,), jnp.float32),
            pltpu.SemaphoreType.DMA,
            pltpu.SemaphoreType.DMA,
        ],
        compiler_params=pltpu.CompilerParams(needs_layout_passes=False),
    )


@jax.jit
def kernel(coords, values, dropped_coords):
    q = dropped_coords - _SHIFT
    qx = q[:, 0].reshape(_NG, 8, 128)
    qy = q[:, 1].reshape(_NG, 8, 128)
    nn_idx = _nn_argmin(qx, qy, coords.T)
    return _make_gather()(values, nn_idx)


# speedup vs baseline: 2.2981x; 1.0054x over previous
import functools

import jax
import jax.numpy as jnp
from jax import lax
from jax.experimental import pallas as pl
from jax.experimental.pallas import tpu as pltpu
from jax.experimental.pallas import tpu_sc as plsc

_SPACING = 0.001
_SHIFT = _SPACING / 2.0

_N_IN = 4096
_C = 128
_N_NEW = 4096

_NG = 4
_UNROLL = 32


def _argmin_body(qx_ref, qy_ref, c_ref, idx_ref):
    shape = (8, 128)
    qx = [qx_ref[g] for g in range(_NG)]
    qy = [qy_ref[g] for g in range(_NG)]
    inf = jnp.full(shape, jnp.inf, jnp.float32)
    zero = jnp.zeros(shape, jnp.int32)
    state = ([inf] * _NG, [zero] * _NG, [inf] * _NG, [zero] * _NG)

    def body(k, st):
        best_a, bidx_a, best_b, bidx_b = st
        best = [list(best_a), list(best_b)]
        bidx = [list(bidx_a), list(bidx_b)]
        j0 = k * _UNROLL
        for u in range(_UNROLL):
            s = u % 2
            j = j0 + u
            cx = jnp.full(shape, c_ref[0, j])
            cy = jnp.full(shape, c_ref[1, j])
            for g in range(_NG):
                dx = qx[g] - cx
                dy = qy[g] - cy
                d2 = dx * dx + dy * dy
                lt = d2 < best[s][g]
                best[s][g] = jnp.where(lt, d2, best[s][g])
                bidx[s][g] = jnp.where(lt, j, bidx[s][g])
        return (best[0], bidx[0], best[1], bidx[1])

    best_a, bidx_a, best_b, bidx_b = lax.fori_loop(
        0, _N_IN // _UNROLL, body, state
    )
    for g in range(_NG):
        take = (best_b[g] < best_a[g]) | (
            (best_b[g] == best_a[g]) & (bidx_b[g] < bidx_a[g])
        )
        idx_ref[g] = jnp.where(take, bidx_b[g], bidx_a[g])


def _nn_argmin(qx, qy, coords_t, interpret=False):
    return pl.pallas_call(
        _argmin_body,
        in_specs=[
            pl.BlockSpec(memory_space=pltpu.VMEM),
            pl.BlockSpec(memory_space=pltpu.VMEM),
            pl.BlockSpec(memory_space=pltpu.SMEM),
        ],
        out_specs=pl.BlockSpec(memory_space=pltpu.VMEM),
        out_shape=jax.ShapeDtypeStruct((_NG, 8, 128), jnp.int32),
        interpret=interpret,
    )(qx, qy, coords_t).reshape(_N_NEW)


_NC, _NS = 2, 16
_NW = _NC * _NS
_R_PER_W = _C // _NW
_L = 16


def _gather_body(values_hbm, idx_hbm, out_hbm, idx_v, rows_v, new_v, sem, lsem):
    wid = lax.axis_index("s") * _NC + lax.axis_index("c")
    row0 = wid * _R_PER_W
    copies = [pltpu.make_async_copy(idx_hbm, idx_v, sem)]
    copies += [
        pltpu.make_async_copy(
            values_hbm.at[row0 + r], rows_v.at[pl.ds(r * _N_IN, _N_IN)], sem
        )
        for r in range(_R_PER_W)
    ]
    for cp in copies:
        cp.start()
    for cp in copies:
        cp.wait()

    left = [
        pltpu.make_async_copy(
            rows_v.at[pl.ds(r * _N_IN, _N_IN)],
            out_hbm.at[row0 + r, pl.ds(0, _N_IN)],
            lsem,
        )
        for r in range(_R_PER_W)
    ]
    for cp in left:
        cp.start()

    gunroll = 4

    def body(k, carry):
        for u in range(gunroll):
            off = (k * gunroll + u) * _L
            ich = idx_v[pl.ds(off, _L)]
            for r in range(_R_PER_W):
                g = plsc.load_gather(rows_v, [ich + (r * _N_IN)])
                new_v[pl.ds(r * _N_IN + off, _L)] = g
        return carry

    lax.fori_loop(0, _N_IN // (_L * gunroll), body, 0)

    outs = [
        pltpu.make_async_copy(
            new_v.at[pl.ds(r * _N_IN, _N_IN)],
            out_hbm.at[row0 + r, pl.ds(_N_IN, _N_IN)],
            sem,
        )
        for r in range(_R_PER_W)
    ]
    for cp in outs:
        cp.start()
    for cp in outs:
        cp.wait()
    for cp in left:
        cp.wait()


@functools.cache
def _make_gather():
    return pl.kernel(
        _gather_body,
        out_type=jax.ShapeDtypeStruct((_C, 2 * _N_IN), jnp.float32),
        mesh=plsc.VectorSubcoreMesh(
            core_axis_name="c", subcore_axis_name="s", num_cores=_NC
        ),
        scratch_types=[
            pltpu.VMEM((_N_NEW,), jnp.int32),
            pltpu.VMEM((_R_PER_W * _N_IN,), jnp.float32),
            pltpu.VMEM((_R_PER_W * _N_IN,), jnp.float32),
            pltpu.SemaphoreType.DMA,
            pltpu.SemaphoreType.DMA,
        ],
        compiler_params=pltpu.CompilerParams(needs_layout_passes=False),
    )


@jax.jit
def kernel(coords, values, dropped_coords):
    q = dropped_coords - _SHIFT
    qx = q[:, 0].reshape(_NG, 8, 128)
    qy = q[:, 1].reshape(_NG, 8, 128)
    nn_idx = _nn_argmin(qx, qy, coords.T)
    return _make_gather()(values, nn_idx)
